# Initial kernel scaffold; baseline (speedup 1.0000x reference)
import jax
import jax.numpy as jnp
from jax.experimental import pallas as pl


def kernel(xyz, scores):
    raise NotImplementedError("write your pallas kernel here")


# speedup vs baseline: 24.8638x; 24.8638x over previous
import functools

import jax
import jax.numpy as jnp
from jax import lax
from jax.experimental import pallas as pl
from jax.experimental.pallas import tpu as pltpu
from jax.experimental.pallas import tpu_sc as plsc

_RADIUS2 = 0.1 * 0.1
_NSAMPLE = 64.0
_MARGIN = 0.5
_N = 8192
_NC = 2
_NS = 16
_NW = _NC * _NS
_ROWS_PER_W = _N // _NW
_L = 16
_GROUPS = _ROWS_PER_W // _L
_UNROLL = 16


def _bf16_round(v):
    u = lax.bitcast_convert_type(v, jnp.uint32)
    lsb = lax.shift_right_logical(u, jnp.uint32(16)) & jnp.uint32(1)
    u = (u + jnp.uint32(0x7FFF) + lsb) & jnp.uint32(0xFFFF0000)
    return lax.bitcast_convert_type(u, jnp.float32)


def _sc_body(x_hbm, y_hbm, z_hbm, s_hbm, out_hbm,
             xv, yv, zv, sv, bxv, byv, bzv, sqv, accv):
    cid = lax.axis_index("c")
    sid = lax.axis_index("s")
    wid = sid * _NC + cid

    pltpu.sync_copy(x_hbm, xv)
    pltpu.sync_copy(y_hbm, yv)
    pltpu.sync_copy(z_hbm, zv)
    pltpu.sync_copy(s_hbm, sv)

    def prep(i, _):
        o = pl.multiple_of(i * _L, 8)
        x = xv[pl.ds(o, _L)]
        y = yv[pl.ds(o, _L)]
        z = zv[pl.ds(o, _L)]
        bxv[pl.ds(o, _L)] = _bf16_round(x)
        byv[pl.ds(o, _L)] = _bf16_round(y)
        bzv[pl.ds(o, _L)] = _bf16_round(z)
        sqv[pl.ds(o, _L)] = x * x + y * y + z * z
        return 0
    lax.fori_loop(0, _N // _L, prep, 0)

    zeros = jnp.zeros((_L,), jnp.float32)

    def group_body(g, acc):
        base = pl.multiple_of(wid * _ROWS_PER_W + g * _L, 8)
        qx = bxv[pl.ds(base, _L)]
        qy = byv[pl.ds(base, _L)]
        qz = bzv[pl.ds(base, _L)]
        qsq = sqv[pl.ds(base, _L)]

        def col_chunk(it, carry):
            cnt, sm, mx = carry
            j0 = pl.multiple_of(it * _UNROLL, 8)
            cxv = bxv[pl.ds(j0, _UNROLL)]
            cyv = byv[pl.ds(j0, _UNROLL)]
            czv = bzv[pl.ds(j0, _UNROLL)]
            cqv = sqv[pl.ds(j0, _UNROLL)]
            csv = sv[pl.ds(j0, _UNROLL)]
            for k in range(_UNROLL):
                dot = qx * cxv[k] + qy * cyv[k] + qz * czv[k]
                d2 = (qsq + cqv[k]) - 2.0 * dot
                take = (d2 < _RADIUS2) & (cnt < _NSAMPLE)
                t = jnp.where(take, csv[k], 0.0)
                sm = sm + t
                mx = jnp.maximum(mx, t)
                cnt = cnt + jnp.where(take, 1.0, 0.0)
            return (cnt, sm, mx)

        cnt, sm, mx = lax.fori_loop(
            0, _N // _UNROLL, col_chunk, (zeros, zeros, zeros))
        mean = sm / jnp.maximum(cnt, 1.0)
        return acc + jnp.maximum(mean - mx + _MARGIN, 0.0)

    accv[...] = lax.fori_loop(0, _GROUPS, group_body, zeros)
    pltpu.sync_copy(accv, out_hbm.at[wid])


@functools.partial(
    pl.kernel,
    mesh=plsc.VectorSubcoreMesh(core_axis_name="c", subcore_axis_name="s"),
    out_type=jax.ShapeDtypeStruct((_NW, _L), jnp.float32),
    scratch_types=[
        pltpu.VMEM((_N,), jnp.float32),
        pltpu.VMEM((_N,), jnp.float32),
        pltpu.VMEM((_N,), jnp.float32),
        pltpu.VMEM((_N,), jnp.float32),
        pltpu.VMEM((_N,), jnp.float32),
        pltpu.VMEM((_N,), jnp.float32),
        pltpu.VMEM((_N,), jnp.float32),
        pltpu.VMEM((_N,), jnp.float32),
        pltpu.VMEM((_L,), jnp.float32),
    ],
)
def _peakness_sc(x_hbm, y_hbm, z_hbm, s_hbm, out_hbm,
                 xv, yv, zv, sv, bxv, byv, bzv, sqv, accv):
    _sc_body(x_hbm, y_hbm, z_hbm, s_hbm, out_hbm,
             xv, yv, zv, sv, bxv, byv, bzv, sqv, accv)


@jax.jit
def kernel(xyz, scores):
    xt = jnp.transpose(xyz)
    x = xt[0] + jnp.float32(0.0)
    y = xt[1] + jnp.float32(0.0)
    z = xt[2] + jnp.float32(0.0)
    partial = _peakness_sc(x, y, z, scores)
    return jnp.sum(partial) / jnp.float32(_N)
